# baseline (speedup 1.0000x reference)
import functools

import jax
import jax.numpy as jnp
from jax import lax
from jax.experimental import pallas as pl
from jax.experimental.pallas import tpu as pltpu
from jax.experimental.pallas import tpu_sc as plsc

_NTAP = 4
_NW = 32
_KC = 128
_EB = 2000
_EB2 = 3200


def _sc_gather(table, idx):
    n, = idx.shape
    _, c = table.shape
    per_w = n // _NW
    nfull = per_w // _KC
    tail = per_w - nfull * _KC
    ns = 6 if nfull % 6 == 0 else 3
    nf = 4 if ns == 6 else 2
    assert nfull % ns == 0 and nfull >= 2 * ns

    mesh = plsc.VectorSubcoreMesh(core_axis_name="c", subcore_axis_name="s")

    @functools.partial(
        pl.kernel,
        mesh=mesh,
        out_type=jax.ShapeDtypeStruct((n, c), table.dtype),
        scratch_types=[
            pltpu.VMEM((per_w,), jnp.int32),
        ] + [pltpu.VMEM((_KC, c), table.dtype)] * ns
          + [pltpu.SemaphoreType.DMA] * (2 * ns),
    )
    def gather_kernel(table_hbm, idx_hbm, out_hbm, idx_v, *bufs):
        rows = bufs[:ns]
        semg = bufs[ns:2 * ns]
        semw = bufs[2 * ns:3 * ns]
        wid = lax.axis_index("s") * 2 + lax.axis_index("c")
        base_w = wid * per_w

        pltpu.sync_copy(idx_hbm.at[pl.ds(base_w, per_w)], idx_v)

        def g_idx(m):
            return idx_v.at[pl.ds(m * _KC, _KC)]

        for m in range(nf):
            pltpu.async_copy(table_hbm.at[g_idx(m)], rows[m], semg[m])

        def step(j, carry):
            for k in range(ns):
                m = ns * j + k
                sl = (k + nf) % ns

                @pl.when(m + nf < nfull)
                def _launch():
                    @pl.when(m + nf >= ns)
                    def _reclaim():
                        pltpu.make_async_copy(
                            rows[sl], out_hbm.at[pl.ds(base_w, _KC)],
                            semw[sl]).wait()
                    pltpu.async_copy(
                        table_hbm.at[g_idx(m + nf)], rows[sl], semg[sl])

                pltpu.make_async_copy(
                    table_hbm.at[g_idx(m)], rows[k], semg[k]).wait()
                pltpu.async_copy(
                    rows[k], out_hbm.at[pl.ds(base_w + m * _KC, _KC)], semw[k])
            return carry

        lax.fori_loop(0, nfull // ns, step, 0)
        for m in range(nfull - ns, nfull):
            pltpu.make_async_copy(
                rows[m % ns], out_hbm.at[pl.ds(base_w, _KC)],
                semw[m % ns]).wait()
        if tail:
            tb = base_w + nfull * _KC
            pltpu.async_copy(
                table_hbm.at[idx_v.at[pl.ds(nfull * _KC, tail)]],
                rows[0].at[pl.ds(0, tail)], semg[0]).wait()
            pltpu.sync_copy(
                rows[0].at[pl.ds(0, tail)], out_hbm.at[pl.ds(tb, tail)])

    return gather_kernel(table, idx)


def _tc_conv_stats(xt, taps_h, wc, y_prev, s1_init, s2_init, half, nhalf):
    e, c = xt.shape
    nbh = (e // nhalf) // _EB
    off = half * nbh

    def body(xt_ref, taps_ref, wc_ref, s1i_ref, s2i_ref, *rest):
        y_ref, s1_ref, s2_ref = rest[-3:]
        i = pl.program_id(0)
        bf = jnp.bfloat16
        x = xt_ref[...].astype(bf)
        a = taps_ref[0]
        bb = taps_ref[1]
        cc = taps_ref[2]
        dd = taps_ref[3]
        h = jnp.concatenate(
            [x, (a + cc).astype(bf), (bb + dd).astype(bf),
             jnp.abs(a - cc).astype(bf), jnp.abs(bb - dd).astype(bf)], axis=1)
        y = jnp.dot(h, wc_ref[...], preferred_element_type=jnp.float32)
        y_ref[...] = y.astype(jnp.bfloat16)

        @pl.when(i == 0)
        def _init():
            s1_ref[...] = s1i_ref[...]
            s2_ref[...] = s2i_ref[...]

        s1_ref[...] += jnp.sum(y, axis=0, keepdims=True)
        s2_ref[...] += jnp.sum(y * y, axis=0, keepdims=True)

    in_specs = [
        pl.BlockSpec((_EB, c), lambda i: (i + off, 0)),
        pl.BlockSpec((_NTAP, _EB, c), lambda i: (0, i, 0)),
        pl.BlockSpec((5 * c, c), lambda i: (0, 0)),
        pl.BlockSpec((1, c), lambda i: (0, 0)),
        pl.BlockSpec((1, c), lambda i: (0, 0)),
    ]
    args = [xt, taps_h, wc, s1_init, s2_init]
    aliases = {}
    if y_prev is not None:
        in_specs.append(pl.BlockSpec(memory_space=pl.ANY))
        args.append(y_prev)
        aliases = {5: 0}

    return pl.pallas_call(
        body,
        grid=(nbh,),
        in_specs=in_specs,
        out_specs=[
            pl.BlockSpec((_EB, c), lambda i: (i + off, 0)),
            pl.BlockSpec((1, c), lambda i: (0, 0)),
            pl.BlockSpec((1, c), lambda i: (0, 0)),
        ],
        out_shape=[
            jax.ShapeDtypeStruct((e, c), jnp.bfloat16),
            jax.ShapeDtypeStruct((1, c), jnp.float32),
            jax.ShapeDtypeStruct((1, c), jnp.float32),
        ],
        input_output_aliases=aliases,
    )(*args)


def _tc_bn_relu(y, scale, shift):
    e, c = y.shape

    def body(y_ref, sc_ref, sh_ref, o_ref):
        yv = y_ref[...].astype(jnp.float32)
        z = jnp.maximum(yv * sc_ref[...] + sh_ref[...], 0.0)
        o_ref[...] = z.T

    return pl.pallas_call(
        body,
        grid=(e // _EB2,),
        in_specs=[
            pl.BlockSpec((_EB2, c), lambda i: (i, 0)),
            pl.BlockSpec((1, c), lambda i: (0, 0)),
            pl.BlockSpec((1, c), lambda i: (0, 0)),
        ],
        out_specs=pl.BlockSpec((c, _EB2), lambda i: (0, i)),
        out_shape=jax.ShapeDtypeStruct((c, e), jnp.float32),
    )(y, scale, shift)


def kernel(x, gemm, W, b, gamma, beta):
    _, c_in, e = x.shape
    c_out = W.shape[0]

    nseg = 4
    eh = e // nseg
    xt = jnp.swapaxes(x[0], 0, 1)
    wc = jnp.transpose(W, (2, 1, 0)).reshape(5 * c_in, c_out).astype(jnp.bfloat16)
    y = None
    s1 = s2 = jnp.zeros((1, c_out), jnp.float32)
    taps = [
        _sc_gather(
            xt, jnp.swapaxes(gemm[0, s * eh:(s + 1) * eh], 0, 1).reshape(-1)
        ).reshape(_NTAP, eh, c_in)
        for s in range(nseg)
    ]
    for s in range(nseg):
        y, s1, s2 = _tc_conv_stats(xt, taps[s], wc, y, s1, s2, s, nseg)

    mean = s1[0] / e
    var = s2[0] / e - mean * mean
    inv = gamma / jnp.sqrt(var + 1e-5)
    scale = inv[None]
    shift = (beta - mean * inv)[None]

    out = _tc_bn_relu(y, scale, shift)[None]
    return (out, gemm)


# speedup vs baseline: 1.0800x; 1.0800x over previous
import functools

import jax
import jax.numpy as jnp
from jax import lax
from jax.experimental import pallas as pl
from jax.experimental.pallas import tpu as pltpu
from jax.experimental.pallas import tpu_sc as plsc

_NTAP = 4
_NW = 32
_KC = 128
_EBF = 3200


def _sc_gather(table, idx):
    n, = idx.shape
    _, c = table.shape
    per_w = n // _NW
    nfull = per_w // _KC
    tail = per_w - nfull * _KC
    ns = 6 if nfull % 6 == 0 else 3
    nf = 4 if ns == 6 else 2
    assert nfull % ns == 0 and nfull >= 2 * ns

    mesh = plsc.VectorSubcoreMesh(core_axis_name="c", subcore_axis_name="s")

    @functools.partial(
        pl.kernel,
        mesh=mesh,
        out_type=jax.ShapeDtypeStruct((n, c), table.dtype),
        scratch_types=[
            pltpu.VMEM((per_w,), jnp.int32),
        ] + [pltpu.VMEM((_KC, c), table.dtype)] * ns
          + [pltpu.SemaphoreType.DMA] * (2 * ns),
    )
    def gather_kernel(table_hbm, idx_hbm, out_hbm, idx_v, *bufs):
        rows = bufs[:ns]
        semg = bufs[ns:2 * ns]
        semw = bufs[2 * ns:3 * ns]
        wid = lax.axis_index("s") * 2 + lax.axis_index("c")
        base_w = wid * per_w

        pltpu.sync_copy(idx_hbm.at[pl.ds(base_w, per_w)], idx_v)

        def g_idx(m):
            return idx_v.at[pl.ds(m * _KC, _KC)]

        for m in range(nf):
            pltpu.async_copy(table_hbm.at[g_idx(m)], rows[m], semg[m])

        def step(j, carry):
            for k in range(ns):
                m = ns * j + k
                sl = (k + nf) % ns

                @pl.when(m + nf < nfull)
                def _launch():
                    @pl.when(m + nf >= ns)
                    def _reclaim():
                        pltpu.make_async_copy(
                            rows[sl], out_hbm.at[pl.ds(base_w, _KC)],
                            semw[sl]).wait()
                    pltpu.async_copy(
                        table_hbm.at[g_idx(m + nf)], rows[sl], semg[sl])

                pltpu.make_async_copy(
                    table_hbm.at[g_idx(m)], rows[k], semg[k]).wait()
                pltpu.async_copy(
                    rows[k], out_hbm.at[pl.ds(base_w + m * _KC, _KC)], semw[k])
            return carry

        lax.fori_loop(0, nfull // ns, step, 0)
        for m in range(nfull - ns, nfull):
            pltpu.make_async_copy(
                rows[m % ns], out_hbm.at[pl.ds(base_w, _KC)],
                semw[m % ns]).wait()
        if tail:
            tb = base_w + nfull * _KC
            pltpu.async_copy(
                table_hbm.at[idx_v.at[pl.ds(nfull * _KC, tail)]],
                rows[0].at[pl.ds(0, tail)], semg[0]).wait()
            pltpu.sync_copy(
                rows[0].at[pl.ds(0, tail)], out_hbm.at[pl.ds(tb, tail)])

    return gather_kernel(table, idx)


def _tc_fused(xt, taps, wc, gamma, beta):
    e, c = xt.shape
    nb = e // _EBF
    inv_e = 1.0 / e

    def body(xt_ref, taps_ref, wc_ref, ga_ref, be_ref, o_ref,
             y_scr, s1, s2, sc_s, sh_s):
        p = pl.program_id(0)
        i = pl.program_id(1)

        @pl.when(p == 0)
        def _conv():
            bf = jnp.bfloat16
            xv = xt_ref[...].astype(bf)
            a = taps_ref[0]
            bb = taps_ref[1]
            cc = taps_ref[2]
            dd = taps_ref[3]
            h = jnp.concatenate(
                [xv, (a + cc).astype(bf), (bb + dd).astype(bf),
                 jnp.abs(a - cc).astype(bf), jnp.abs(bb - dd).astype(bf)],
                axis=1)
            y = jnp.dot(h, wc_ref[...], preferred_element_type=jnp.float32)
            y_scr[pl.ds(i * _EBF, _EBF), :] = y.astype(bf)

            @pl.when(i == 0)
            def _init():
                s1[...] = jnp.zeros_like(s1)
                s2[...] = jnp.zeros_like(s2)

            s1[...] += jnp.sum(y, axis=0, keepdims=True)
            s2[...] += jnp.sum(y * y, axis=0, keepdims=True)

        @pl.when(p == 1)
        def _bn():
            @pl.when(i == 0)
            def _finalize():
                mean = s1[...] * inv_e
                var = s2[...] * inv_e - mean * mean
                inv = ga_ref[...] * jax.lax.rsqrt(var + 1e-5)
                sc_s[...] = inv
                sh_s[...] = be_ref[...] - mean * inv

            yv = y_scr[pl.ds(i * _EBF, _EBF), :].astype(jnp.float32)
            z = jnp.maximum(yv * sc_s[...] + sh_s[...], 0.0)
            o_ref[...] = z.T

    return pl.pallas_call(
        body,
        grid=(2, nb),
        in_specs=[
            pl.BlockSpec((_EBF, c), lambda p, i: (i * (1 - p), 0)),
            pl.BlockSpec((_NTAP, _EBF, c), lambda p, i: (0, i * (1 - p), 0)),
            pl.BlockSpec((5 * c, c), lambda p, i: (0, 0)),
            pl.BlockSpec((1, c), lambda p, i: (0, 0)),
            pl.BlockSpec((1, c), lambda p, i: (0, 0)),
        ],
        out_specs=pl.BlockSpec((c, _EBF), lambda p, i: (0, i * p)),
        out_shape=jax.ShapeDtypeStruct((c, e), jnp.float32),
        scratch_shapes=[
            pltpu.VMEM((e, c), jnp.bfloat16),
            pltpu.VMEM((1, c), jnp.float32),
            pltpu.VMEM((1, c), jnp.float32),
            pltpu.VMEM((1, c), jnp.float32),
            pltpu.VMEM((1, c), jnp.float32),
        ],
        compiler_params=pltpu.CompilerParams(
            vmem_limit_bytes=110 * 1024 * 1024),
    )(xt, taps, wc, gamma[None], beta[None])


def kernel(x, gemm, W, b, gamma, beta):
    _, c_in, e = x.shape
    c_out = W.shape[0]

    xt = jnp.swapaxes(x[0], 0, 1)
    idx = jnp.swapaxes(gemm[0], 0, 1).reshape(-1)
    taps = _sc_gather(xt, idx).reshape(_NTAP, e, c_in)

    wc = jnp.transpose(W, (2, 1, 0)).reshape(5 * c_in, c_out).astype(jnp.bfloat16)
    out = _tc_fused(xt, taps, wc, gamma, beta)[None]
    return (out, gemm)
